# baseline (speedup 1.0000x reference)
import jax
import jax.numpy as jnp
from jax import lax
from jax.experimental import pallas as pl
from jax.experimental.pallas import tpu as pltpu
from jax.experimental.pallas import tpu_sc as plsc
import functools

B = 4096
SEQ = 200
D = 128
DP = D // 2
NC = 2
NS = 16
NW = NC * NS
BPW = B // NW
C0 = 128
C1 = SEQ - C0
RB = 8


def _make_bag_kernel():
    mesh = plsc.VectorSubcoreMesh(core_axis_name="c", subcore_axis_name="s")

    @functools.partial(
        pl.kernel,
        mesh=mesh,
        out_type=jax.ShapeDtypeStruct((B, D), jnp.float32),
        scratch_types=[
            pltpu.VMEM((2, IG, SEQ), jnp.int32),
            pltpu.VMEM((RR, C0, D), jnp.float32),
            pltpu.VMEM((RR, C1, D), jnp.float32),
            pltpu.VMEM((STG, D), jnp.float32),
            [pltpu.SemaphoreType.DMA] * RR,
            [pltpu.SemaphoreType.DMA] * RR,
            [pltpu.SemaphoreType.DMA] * 2,
        ],
    )
    def bag(x_hbm, table_hbm, rep_hbm, idxg, rowsE, rowsO, stage_v,
            esems, osems, psems):
        sid = lax.axis_index("s")
        wid = sid * NC + lax.axis_index("c")
        base = wid * BPW

        def issue(slot, local, k4):
            pltpu.async_copy(table_hbm.at[idxg.at[slot, local, pl.ds(0, C0)]],
                             rowsE.at[k4], esems[k4])
            pltpu.async_copy(table_hbm.at[idxg.at[slot, local, pl.ds(C0, C1)]],
                             rowsO.at[k4], osems[k4])

        def accumulate(rows_v, k4, n, accs):
            def inner(i, a):
                new = []
                for j in range(8):
                    new.append(a[j] + rows_v[k4, i, pl.ds(j * 16, 16)])
                return tuple(new)
            return lax.fori_loop(0, n, inner, accs, unroll=4)

        def row_step(slot, local, k4, srow):
            pltpu.make_async_copy(
                table_hbm.at[idxg.at[slot, local, pl.ds(0, C0)]],
                rowsE.at[k4], esems[k4]).wait()
            zero = jnp.zeros((16,), jnp.float32)
            accs = accumulate(rowsE, k4, C0, (zero,) * 8)
            pltpu.make_async_copy(
                table_hbm.at[idxg.at[slot, local, pl.ds(C0, C1)]],
                rowsO.at[k4], osems[k4]).wait()
            accs = accumulate(rowsO, k4, C1, accs)
            for j in range(8):
                stage_v[srow, pl.ds(j * 16, 16)] = accs[j]

        def pref_wait(slot):
            pltpu.make_async_copy(
                x_hbm.at[pl.ds(base, IG)], idxg.at[slot], psems[slot]).wait()

        pltpu.sync_copy(x_hbm.at[pl.ds(base, IG)], idxg.at[0])
        pltpu.async_copy(x_hbm.at[pl.ds(base + IG, IG)], idxg.at[1], psems[1])
        for k in range(RR):
            issue(0, k, k)

        def body(i, carry):
            pref_wait(1)
            for k in range(IG):
                row_step(0, k, k % RR, k)
                if k < IG - RR:
                    issue(0, k + RR, k % RR)
                else:
                    issue(1, k + RR - IG, k % RR)
            @pl.when(i < NBODY - 1)
            def _():
                off = pl.multiple_of(base + (2 * i + 2) * IG, IG)
                pltpu.async_copy(x_hbm.at[pl.ds(off, IG)], idxg.at[0],
                                 psems[0])
            for k in range(IG):
                row_step(1, k, k % RR, IG + k)
                if k < IG - RR:
                    issue(1, k + RR, k % RR)
                elif k == IG - RR:
                    @pl.when(i < NBODY - 1)
                    def _():
                        pref_wait(0)
                        issue(0, 0, 0)
                else:
                    @pl.when(i < NBODY - 1)
                    def _(k=k):
                        issue(0, k + RR - IG, k % RR)
            @pl.when(i < NBODY - 1)
            def _():
                off1 = pl.multiple_of(base + (2 * i + 3) * IG, IG)
                pltpu.async_copy(x_hbm.at[pl.ds(off1, IG)], idxg.at[1],
                                 psems[1])
            off = pl.multiple_of(base + i * STG, STG)
            pltpu.sync_copy(stage_v, rep_hbm.at[pl.ds(off, STG)])
            return carry

        lax.fori_loop(0, NBODY, body, 0)

    return bag


def _mlp_body(rep_ref, len_ref, w1_ref, b1_ref, w2_ref, b2_ref, out_ref):
    rep = rep_ref[...] / len_ref[...]
    h = jnp.maximum(
        jnp.dot(rep, w1_ref[...].T, preferred_element_type=jnp.float32)
        + b1_ref[...], 0.0)
    out_ref[...] = (
        jnp.dot(h, w2_ref[...].T, preferred_element_type=jnp.float32)
        + b2_ref[...])


def kernel(x, lengths, table, W1, b1, W2, b2):
    table_bf = table.astype(jnp.bfloat16)
    table_i32 = lax.bitcast_convert_type(
        table_bf.reshape(table.shape[0], DP, 2), jnp.int32)

    rep = _make_bag_kernel()(x_r, table_i32)

    ar = jnp.arange(16, dtype=jnp.int32)
    block = jnp.concatenate([2 * ar, 2 * ar + 1])
    perm = (jnp.arange(4, dtype=jnp.int32)[:, None] * 32
            + block[None, :]).reshape(-1)
    W1p = W1[:, perm]

    hidden = W1.shape[0]
    out_size = W2.shape[0]
    blk = B
    grid = (B // blk,)
    logits = pl.pallas_call(
        _mlp_body,
        grid=grid,
        in_specs=[
            pl.BlockSpec((blk, D), lambda i: (i, 0)),
            pl.BlockSpec((blk, 1), lambda i: (i, 0)),
            pl.BlockSpec((hidden, D), lambda i: (0, 0)),
            pl.BlockSpec((1, hidden), lambda i: (0, 0)),
            pl.BlockSpec((out_size, hidden), lambda i: (0, 0)),
            pl.BlockSpec((1, out_size), lambda i: (0, 0)),
        ],
        out_specs=pl.BlockSpec((blk, out_size), lambda i: (i, 0)),
        out_shape=jax.ShapeDtypeStruct((B, out_size), jnp.float32),
    )(rep, lengths.astype(jnp.float32).reshape(B, 1),
      W1p, b1.reshape(1, hidden), W2, b2.reshape(1, out_size))
    return logits


# speedup vs baseline: 1.0439x; 1.0439x over previous
import jax
import jax.numpy as jnp
from jax import lax
from jax.experimental import pallas as pl
from jax.experimental.pallas import tpu as pltpu
from jax.experimental.pallas import tpu_sc as plsc
import functools

B = 4096
SEQ = 200
D = 128
DP = D // 2
NC = 2
NS = 16
NW = NC * NS
BPW = B // NW
C0 = 128
C1 = SEQ - C0
RB = 8


def _make_bag_kernel():
    mesh = plsc.VectorSubcoreMesh(core_axis_name="c", subcore_axis_name="s")

    @functools.partial(
        pl.kernel,
        mesh=mesh,
        out_type=jax.ShapeDtypeStruct((B, D), jnp.float32),
        scratch_types=[
            pltpu.VMEM((BPW, SEQ), jnp.int32),
            pltpu.VMEM((RR, C0, D), jnp.float32),
            pltpu.VMEM((RR, C1, D), jnp.float32),
            pltpu.VMEM((STG, D), jnp.float32),
            [pltpu.SemaphoreType.DMA] * RR,
            [pltpu.SemaphoreType.DMA] * RR,
        ],
    )
    def bag(x_hbm, table_hbm, rep_hbm, idx_v, rowsE, rowsO, stage_v,
            esems, osems):
        sid = lax.axis_index("s")
        wid = sid * NC + lax.axis_index("c")
        base = wid * BPW
        pltpu.sync_copy(x_hbm.at[pl.ds(base, BPW)], idx_v)

        def issue(r, k):
            pltpu.async_copy(table_hbm.at[idx_v.at[r, pl.ds(0, C0)]],
                             rowsE.at[k], esems[k])
            pltpu.async_copy(table_hbm.at[idx_v.at[r, pl.ds(C0, C1)]],
                             rowsO.at[k], osems[k])

        def accumulate(rows_v, k, n, accs):
            def inner(i, a):
                new = []
                for j in range(8):
                    new.append(a[j] + rows_v[k, i, pl.ds(j * 16, 16)])
                return tuple(new)
            return lax.fori_loop(0, n, inner, accs, unroll=8)

        def issueE(r, k):
            pltpu.async_copy(table_hbm.at[idx_v.at[r, pl.ds(0, C0)]],
                             rowsE.at[k], esems[k])

        def issueO(r, k):
            pltpu.async_copy(table_hbm.at[idx_v.at[r, pl.ds(C0, C1)]],
                             rowsO.at[k], osems[k])

        def row_step(r, k, guard):
            pltpu.make_async_copy(
                table_hbm.at[idx_v.at[r, pl.ds(0, C0)]],
                rowsE.at[k], esems[k]).wait()
            zero = jnp.zeros((16,), jnp.float32)
            accs = accumulate(rowsE, k, C0, (zero,) * 8)
            if guard == "traced":
                @pl.when(r + RR < BPW)
                def _():
                    issueE(r + RR, k)
            elif guard + RR < BPW:
                issueE(r + RR, k)
            pltpu.make_async_copy(
                table_hbm.at[idx_v.at[r, pl.ds(C0, C1)]],
                rowsO.at[k], osems[k]).wait()
            accs = accumulate(rowsO, k, C1, accs)
            if guard == "traced":
                @pl.when(r + RR < BPW)
                def _():
                    issueO(r + RR, k)
            elif guard + RR < BPW:
                issueO(r + RR, k)
            sr = lax.rem(r, STG)
            for j in range(8):
                stage_v[sr, pl.ds(j * 16, 16)] = accs[j]

            @pl.when(sr == STG - 1)
            def _():
                off = pl.multiple_of(base + r - (STG - 1), STG)
                pltpu.sync_copy(stage_v, rep_hbm.at[pl.ds(off, STG)])

        for k in range(RR):
            issue(k, k)

        def body(g, carry):
            for k in range(RR):
                row_step(g * RR + k, k, "traced")
            return carry

        lax.fori_loop(0, BPW // RR, body, 0)
        for k in range(BPW % RR):
            r = (BPW // RR) * RR + k
            row_step(r, k, r)

    return bag


def _mlp_body(rep_ref, len_ref, w1_ref, b1_ref, w2_ref, b2_ref, out_ref):
    rep = rep_ref[...] / len_ref[...]
    h = jnp.maximum(
        jnp.dot(rep, w1_ref[...].T, preferred_element_type=jnp.float32)
        + b1_ref[...], 0.0)
    out_ref[...] = (
        jnp.dot(h, w2_ref[...].T, preferred_element_type=jnp.float32)
        + b2_ref[...])


def kernel(x, lengths, table, W1, b1, W2, b2):
    table_bf = table.astype(jnp.bfloat16)
    table_i32 = lax.bitcast_convert_type(
        table_bf.reshape(table.shape[0], DP, 2), jnp.int32)

    rep = _make_bag_kernel()(x_r, table_i32)

    ar = jnp.arange(16, dtype=jnp.int32)
    block = jnp.concatenate([2 * ar, 2 * ar + 1])
    perm = (jnp.arange(4, dtype=jnp.int32)[:, None] * 32
            + block[None, :]).reshape(-1)
    W1p = W1[:, perm]

    hidden = W1.shape[0]
    out_size = W2.shape[0]
    blk = B
    grid = (B // blk,)
    logits = pl.pallas_call(
        _mlp_body,
        grid=grid,
        in_specs=[
            pl.BlockSpec((blk, D), lambda i: (i, 0)),
            pl.BlockSpec((blk, 1), lambda i: (i, 0)),
            pl.BlockSpec((hidden, D), lambda i: (0, 0)),
            pl.BlockSpec((1, hidden), lambda i: (0, 0)),
            pl.BlockSpec((out_size, hidden), lambda i: (0, 0)),
            pl.BlockSpec((1, out_size), lambda i: (0, 0)),
        ],
        out_specs=pl.BlockSpec((blk, out_size), lambda i: (i, 0)),
        out_shape=jax.ShapeDtypeStruct((B, out_size), jnp.float32),
    )(rep, lengths.astype(jnp.float32).reshape(B, 1),
      W1p, b1.reshape(1, hidden), W2, b2.reshape(1, out_size))
    return logits
